# baseline (speedup 1.0000x reference)
import functools

import jax
import jax.numpy as jnp
from jax import lax
from jax.experimental import pallas as pl
from jax.experimental.pallas import tpu as pltpu
from jax.experimental.pallas import tpu_sc as plsc

B = 4096
S = 200
D = 32
NC = 2
NS = 16
NW = NC * NS
ROWS_PER_W = B // NW
ROWS_PER_CHUNK = 4
NCHUNK = ROWS_PER_W // ROWS_PER_CHUNK


@functools.partial(
    pl.kernel,
    out_type=jax.ShapeDtypeStruct((B, S, D), jnp.float32),
    mesh=plsc.VectorSubcoreMesh(core_axis_name="c", subcore_axis_name="s"),
    scratch_types=[
        pltpu.VMEM((ROWS_PER_CHUNK, S), jnp.int32),
        pltpu.VMEM((ROWS_PER_CHUNK, S, D), jnp.float32),
        pltpu.VMEM((S, D), jnp.float32),
        pltpu.SemaphoreType.DMA,
    ],
    compiler_params=pltpu.CompilerParams(use_tc_tiling_on_sc=False),
)
def _emb_lookup(x_hbm, emb_hbm, pos_hbm, out_hbm, idx_v, rows_v, pos_v, sem):
    wid = lax.axis_index("s") * NC + lax.axis_index("c")
    base = wid * ROWS_PER_W
    pltpu.sync_copy(pos_hbm, pos_v)

    def chunk_body(c, _):
        row = base + c * ROWS_PER_CHUNK
        pltpu.sync_copy(x_hbm.at[pl.ds(row, ROWS_PER_CHUNK)], idx_v)
        copies = [
            pltpu.async_copy(emb_hbm.at[idx_v.at[i]], rows_v.at[i], sem)
            for i in range(ROWS_PER_CHUNK)
        ]
        for cp in copies:
            cp.wait()

        def row_body(r, _):
            p0 = pos_v[r, pl.ds(0, 16)]
            p1 = pos_v[r, pl.ds(16, 16)]
            for i in range(ROWS_PER_CHUNK):
                rows_v[i, r, pl.ds(0, 16)] = rows_v[i, r, pl.ds(0, 16)] + p0
                rows_v[i, r, pl.ds(16, 16)] = rows_v[i, r, pl.ds(16, 16)] + p1
            return 0

        lax.fori_loop(0, S, row_body, 0)
        pltpu.sync_copy(rows_v, out_hbm.at[pl.ds(row, ROWS_PER_CHUNK)])
        return 0

    lax.fori_loop(0, NCHUNK, chunk_body, 0)


def kernel(x, embedding, pos_embedding):
    return _emb_lookup(x, embedding, pos_embedding)
